# baseline (speedup 1.0000x reference)
import functools

import jax
import jax.numpy as jnp
from jax.experimental import pallas as pl
from jax.experimental.pallas import tpu as pltpu

B, S, H = 2, 2048, 2048
E = 8
TOPK = 2
I = 1024

M = 1024
IC = 512
C = I // IC


def _mlp_body(x_ref, wr_ref, wg_ref, wu_ref, wd_ref, out_ref, w_scr):
    e = pl.program_id(1)
    c = pl.program_id(2)
    first = jnp.logical_and(e == 0, c == 0)

    @pl.when(first)
    def _():
        logits = jnp.dot(x_ref[...], wr_ref[...].T,
                         preferred_element_type=jnp.float32)
        mx = jnp.max(logits, axis=-1, keepdims=True)
        ex = jnp.exp(logits - mx)
        v1 = jnp.max(ex, axis=-1, keepdims=True)
        lane = jax.lax.broadcasted_iota(jnp.int32, ex.shape, 1)
        idx1 = jnp.min(jnp.where(ex == v1, lane, E), axis=-1, keepdims=True)
        v2 = jnp.max(jnp.where(lane == idx1, -jnp.inf, ex), axis=-1,
                     keepdims=True)
        denom = v1 + v2
        w_scr[...] = jnp.where(lane == 0, v1 / denom,
                               jnp.where(lane == 1, v2 / denom, 0.0))

    gate = jnp.dot(x_ref[...], wg_ref[0], preferred_element_type=jnp.float32)
    up = jnp.dot(x_ref[...], wu_ref[0], preferred_element_type=jnp.float32)
    lane = jax.lax.broadcasted_iota(jnp.int32, (M, E), 1)
    wcol = jnp.sum(jnp.where(lane == e, w_scr[...], 0.0), axis=1,
                   keepdims=True)
    hid = gate * jax.nn.sigmoid(gate) * (up * wcol)
    part = jnp.dot(hid.astype(jnp.bfloat16), wd_ref[0],
                   preferred_element_type=jnp.float32)

    @pl.when(first)
    def _():
        out_ref[...] = part

    @pl.when(jnp.logical_not(first))
    def _():
        out_ref[...] += part


@jax.jit
def kernel(hidden_states, router_weight, gate_up_weights, down_weights):
    b, s, h = hidden_states.shape
    n = b * s
    hflat = hidden_states.reshape(n, h)

    x16 = hflat.astype(jnp.bfloat16)
    wr16 = router_weight.astype(jnp.bfloat16)
    gu16 = gate_up_weights[:TOPK].astype(jnp.bfloat16)
    dn16 = down_weights[:TOPK].astype(jnp.bfloat16)

    out = pl.pallas_call(
        _mlp_body,
        grid=(n // M, TOPK, C),
        in_specs=[
            pl.BlockSpec((M, h), lambda t, e, c: (t, 0)),
            pl.BlockSpec((E, h), lambda t, e, c: (0, 0)),
            pl.BlockSpec((1, h, IC), lambda t, e, c: (e, 0, c)),
            pl.BlockSpec((1, h, IC), lambda t, e, c: (e, 0, C + c)),
            pl.BlockSpec((1, IC, h), lambda t, e, c: (e, c, 0)),
        ],
        out_specs=pl.BlockSpec((M, h), lambda t, e, c: (t, 0)),
        out_shape=jax.ShapeDtypeStruct((n, h), jnp.float32),
        scratch_shapes=[pltpu.VMEM((M, E), jnp.float32)],
    )(x16, wr16, gu16, gu16, dn16)

    return out.reshape(b, s, h)


# speedup vs baseline: 1.1128x; 1.1128x over previous
import jax
import jax.numpy as jnp
from jax.experimental import pallas as pl

B, S, H = 2, 2048, 2048
E = 8
TOPK = 2
I = 1024

M = 512


def _mlp_body(x_ref, wr_ref, wgu_ref, wd_ref, out_ref):
    x32 = x_ref[...]
    logits = jnp.dot(x32, wr_ref[...].T, preferred_element_type=jnp.float32)
    mx = jnp.max(logits, axis=-1, keepdims=True)
    ex = jnp.exp(logits - mx)
    v1 = jnp.max(ex, axis=-1, keepdims=True)
    lane = jax.lax.broadcasted_iota(jnp.int32, ex.shape, 1)
    idx1 = jnp.min(jnp.where(ex == v1, lane, E), axis=-1, keepdims=True)
    v2 = jnp.max(jnp.where(lane == idx1, -jnp.inf, ex), axis=-1, keepdims=True)
    denom = v1 + v2

    x16 = x32.astype(jnp.bfloat16)
    acc = None
    for e in range(TOPK):
        gate = jnp.dot(x16, wgu_ref[e, :, :I], preferred_element_type=jnp.float32)
        up = jnp.dot(x16, wgu_ref[e, :, I:], preferred_element_type=jnp.float32)
        w = (v1 if e == 0 else v2) / denom
        hid = gate * jax.nn.sigmoid(gate) * (up * w)
        part = jnp.dot(hid.astype(jnp.bfloat16), wd_ref[e],
                       preferred_element_type=jnp.float32)
        acc = part if acc is None else acc + part
    out_ref[...] = acc


@jax.jit
def kernel(hidden_states, router_weight, gate_up_weights, down_weights):
    b, s, h = hidden_states.shape
    n = b * s
    hflat = hidden_states.reshape(n, h)

    gu16 = gate_up_weights[:TOPK].astype(jnp.bfloat16)
    dn16 = down_weights[:TOPK].astype(jnp.bfloat16)

    out = pl.pallas_call(
        _mlp_body,
        grid=(n // M,),
        in_specs=[
            pl.BlockSpec((M, h), lambda t: (t, 0)),
            pl.BlockSpec((E, h), lambda t: (0, 0)),
            pl.BlockSpec((TOPK, h, 2 * I), lambda t: (0, 0, 0)),
            pl.BlockSpec((TOPK, I, h), lambda t: (0, 0, 0)),
        ],
        out_specs=pl.BlockSpec((M, h), lambda t: (t, 0)),
        out_shape=jax.ShapeDtypeStruct((n, h), jnp.float32),
    )(hflat, router_weight, gu16, dn16)

    return out.reshape(b, s, h)
